# baseline (speedup 1.0000x reference)
import functools

import jax
import jax.numpy as jnp
from jax import lax
from jax.experimental import pallas as pl
from jax.experimental.pallas import tpu as pltpu
from jax.experimental.pallas import tpu_sc as plsc

N_NODES = 10000
H_DIM = 128

NUM_CORES = 2
NUM_SUBCORES = 16
NUM_WORKERS = NUM_CORES * NUM_SUBCORES

CHUNK = 128

N_PAD = 10368
N_CHUNKS = N_PAD // CHUNK


def _edge_padding(num_edges):
  per_round = 2 * NUM_WORKERS * CHUNK
  rounds = -(-num_edges // per_round)
  if rounds % 2 == 0:
    rounds += 1
  return 2 * rounds * NUM_WORKERS * CHUNK, rounds


def _sc_mesh():
  return plsc.VectorSubcoreMesh(
      core_axis_name="c", subcore_axis_name="s",
      num_cores=NUM_CORES, num_subcores=NUM_SUBCORES)


def _make_sc_gather2(ea_pad, chunks_per_worker):
  per_worker = chunks_per_worker * CHUNK
  assert chunks_per_worker % 2 == 1

  @functools.partial(
      pl.kernel,
      mesh=_sc_mesh(),
      out_type=(
          jax.ShapeDtypeStruct((ea_pad, H_DIM), jnp.float32),
          jax.ShapeDtypeStruct((ea_pad, H_DIM), jnp.float32),
      ),
      scratch_types=[
          pltpu.VMEM((chunks_per_worker, CHUNK), jnp.int32),
          pltpu.VMEM((chunks_per_worker, CHUNK), jnp.int32),
          pltpu.VMEM((CHUNK, H_DIM), jnp.float32),
          pltpu.VMEM((CHUNK, H_DIM), jnp.float32),
          pltpu.VMEM((CHUNK, H_DIM), jnp.float32),
          pltpu.VMEM((CHUNK, H_DIM), jnp.float32),
          pltpu.SemaphoreType.DMA,
          pltpu.SemaphoreType.DMA,
          pltpu.SemaphoreType.DMA,
          pltpu.SemaphoreType.DMA,
      ],
  )
  def gather2(h_hbm, dst_hbm, src_hbm, out_d, out_s, idx_d, idx_s,
              rows_ad, rows_as, rows_bd, rows_bs,
              sem_ad, sem_as, sem_bd, sem_bs):
    wid = lax.axis_index("s") * NUM_CORES + lax.axis_index("c")
    pltpu.sync_copy(dst_hbm.at[wid], idx_d)
    pltpu.sync_copy(src_hbm.at[wid], idx_s)

    def fire(c, rows, sem, idx):
      pltpu.async_copy(h_hbm.at[idx.at[c]], rows, sem)

    def wait(c, rows, sem, idx):
      pltpu.make_async_copy(h_hbm.at[idx.at[c]], rows, sem).wait()

    def write(c, rows, out):
      pltpu.sync_copy(
          rows, out.at[pl.ds(wid * per_worker + c * CHUNK, CHUNK)])

    fire(0, rows_ad, sem_ad, idx_d)
    fire(0, rows_as, sem_as, idx_s)

    def body(t, _):
      c0 = 2 * t
      c1 = c0 + 1
      c2 = c0 + 2
      fire(c1, rows_bd, sem_bd, idx_d)
      fire(c1, rows_bs, sem_bs, idx_s)
      wait(c0, rows_ad, sem_ad, idx_d)
      wait(c0, rows_as, sem_as, idx_s)
      write(c0, rows_ad, out_d)
      write(c0, rows_as, out_s)
      fire(c2, rows_ad, sem_ad, idx_d)
      fire(c2, rows_as, sem_as, idx_s)
      wait(c1, rows_bd, sem_bd, idx_d)
      wait(c1, rows_bs, sem_bs, idx_s)
      write(c1, rows_bd, out_d)
      write(c1, rows_bs, out_s)
      return 0

    lax.fori_loop(0, (chunks_per_worker - 1) // 2, body, 0)
    c_last = chunks_per_worker - 1
    wait(c_last, rows_ad, sem_ad, idx_d)
    wait(c_last, rows_as, sem_as, idx_s)
    write(c_last, rows_ad, out_d)
    write(c_last, rows_as, out_s)

  return gather2


def _make_sc_scatter_add(ea_pad, chunks_per_worker):
  per_worker = chunks_per_worker * CHUNK

  @functools.partial(
      pl.kernel,
      mesh=_sc_mesh(),
      out_type=(
          jax.ShapeDtypeStruct((N_PAD, H_DIM), jnp.float32),
          jax.ShapeDtypeStruct((N_PAD, H_DIM), jnp.float32),
      ),
      scratch_types=[
          pltpu.VMEM((chunks_per_worker, CHUNK), jnp.int32),
          pltpu.VMEM((CHUNK, H_DIM), jnp.float32),
          pltpu.VMEM((CHUNK, H_DIM), jnp.float32),
          pltpu.VMEM_SHARED((N_PAD, H_DIM), jnp.float32),
          pltpu.SemaphoreType.DMA,
          pltpu.SemaphoreType.DMA,
      ],
  )
  def scatter_add(w_hbm, dst_hbm, out0, out1, idx_v, rows_a, rows_b, acc_sh,
                  sem_a, sem_b):
    cid = lax.axis_index("c")
    sid = lax.axis_index("s")
    wid = sid * NUM_CORES + cid

    zvec = jnp.zeros((16,), jnp.float32)

    def zero_body(i, _):
      rows_a[i // 8, pl.ds((i % 8) * 16, 16)] = zvec
      return 0

    lax.fori_loop(0, CHUNK * 8, zero_body, 0)

    def zero_acc(t, _):
      j = sid + t * NUM_SUBCORES

      @pl.when(j < N_CHUNKS)
      def _():
        pltpu.sync_copy(rows_a, acc_sh.at[pl.ds(j * CHUNK, CHUNK)])

      return 0

    lax.fori_loop(0, -(-N_CHUNKS // NUM_SUBCORES), zero_acc, 0)
    plsc.subcore_barrier()

    pltpu.sync_copy(dst_hbm.at[wid], idx_v)

    def fire(c, rows, sem):
      base = wid * per_worker + c * CHUNK
      pltpu.async_copy(w_hbm.at[pl.ds(base, CHUNK)], rows, sem)

    def wait(c, rows, sem):
      base = wid * per_worker + c * CHUNK
      pltpu.make_async_copy(w_hbm.at[pl.ds(base, CHUNK)], rows, sem).wait()

    def add(c, rows):
      pltpu.sync_copy(rows, acc_sh.at[idx_v.at[c]], add=True)

    fire(0, rows_a, sem_a)

    def body(t, _):
      c0 = 2 * t
      c1 = c0 + 1
      c2 = c0 + 2
      fire(c1, rows_b, sem_b)
      wait(c0, rows_a, sem_a)
      add(c0, rows_a)
      fire(c2, rows_a, sem_a)
      wait(c1, rows_b, sem_b)
      add(c1, rows_b)
      return 0

    lax.fori_loop(0, (chunks_per_worker - 1) // 2, body, 0)
    c_last = chunks_per_worker - 1
    wait(c_last, rows_a, sem_a)
    add(c_last, rows_a)
    plsc.subcore_barrier()

    def write_out(t, _):
      j = sid + t * NUM_SUBCORES

      @pl.when(j < N_CHUNKS)
      def _():
        pltpu.sync_copy(acc_sh.at[pl.ds(j * CHUNK, CHUNK)], rows_a)

        @pl.when(cid == 0)
        def _():
          pltpu.sync_copy(rows_a, out0.at[pl.ds(j * CHUNK, CHUNK)])

        @pl.when(cid == 1)
        def _():
          pltpu.sync_copy(rows_a, out1.at[pl.ds(j * CHUNK, CHUNK)])

      return 0

    lax.fori_loop(0, -(-N_CHUNKS // NUM_SUBCORES), write_out, 0)

  return scatter_add


def _ln_relu(u, g, b):
  mu = jnp.mean(u, axis=-1, keepdims=True)
  var = jnp.mean((u - mu) ** 2, axis=-1, keepdims=True)
  return jax.nn.relu((u - mu) * lax.rsqrt(var + 1e-5) * g + b)


def _tc_input_body(x_ref, w_ref, b_ref, g_ref, bb_ref, o_ref):
  u = jnp.dot(x_ref[...], w_ref[...], preferred_element_type=jnp.float32)
  o_ref[...] = _ln_relu(u + b_ref[...], g_ref[...], bb_ref[...])


def _tc_input(x, w_in, b_in, g, b):
  blk = 1000
  grid = (N_NODES // blk,)
  return pl.pallas_call(
      _tc_input_body,
      grid=grid,
      in_specs=[
          pl.BlockSpec((blk, H_DIM), lambda i: (i, 0)),
          pl.BlockSpec((H_DIM, H_DIM), lambda i: (0, 0)),
          pl.BlockSpec((1, H_DIM), lambda i: (0, 0)),
          pl.BlockSpec((1, H_DIM), lambda i: (0, 0)),
          pl.BlockSpec((1, H_DIM), lambda i: (0, 0)),
      ],
      out_specs=pl.BlockSpec((blk, H_DIM), lambda i: (i, 0)),
      out_shape=jax.ShapeDtypeStruct((N_NODES, H_DIM), jnp.float32),
  )(x, w_in, b_in.reshape(1, -1), g.reshape(1, -1), b.reshape(1, -1))


def _tc_edge_body(num_edges, blk, offset, hd_ref, hs_ref, w1d_ref, w1s_ref,
                  w1a_ref, b1_ref, w2_ref, b2_ref, wmsg_ref, o_ref):
  hd = hd_ref[...].astype(jnp.float32)
  hs = hs_ref[...].astype(jnp.float32)
  d = jnp.abs(hd - hs)
  hidden = jnp.dot(hd, w1d_ref[...], preferred_element_type=jnp.float32)
  hidden += jnp.dot(hs, w1s_ref[...], preferred_element_type=jnp.float32)
  hidden += jnp.dot(d, w1a_ref[...], preferred_element_type=jnp.float32)
  hidden = jax.nn.relu(hidden + b1_ref[...])
  s = jnp.sum(hidden * w2_ref[...], axis=-1, keepdims=True) + b2_ref[0, :1]
  score = jax.nn.sigmoid(s)
  msg = jnp.dot(hs, wmsg_ref[...], preferred_element_type=jnp.float32)
  rows = (offset + pl.program_id(0) * blk
          + lax.broadcasted_iota(jnp.int32, (blk, 1), 0))
  o_ref[...] = jnp.where(rows < num_edges, score * msg, 0.0)


def _tc_edge(hd, hs, w1, b1, w2, b2, wmsg, num_edges, offset):
  ea_pad = hd.shape[0]
  blk = 512
  grid = (ea_pad // blk,)
  w1d = w1[:H_DIM]
  w1s = w1[H_DIM:2 * H_DIM]
  w1a = w1[2 * H_DIM:]
  return pl.pallas_call(
      functools.partial(_tc_edge_body, num_edges, blk, offset),
      grid=grid,
      in_specs=[
          pl.BlockSpec((blk, H_DIM), lambda i: (i, 0)),
          pl.BlockSpec((blk, H_DIM), lambda i: (i, 0)),
          pl.BlockSpec((H_DIM, H_DIM), lambda i: (0, 0)),
          pl.BlockSpec((H_DIM, H_DIM), lambda i: (0, 0)),
          pl.BlockSpec((H_DIM, H_DIM), lambda i: (0, 0)),
          pl.BlockSpec((1, H_DIM), lambda i: (0, 0)),
          pl.BlockSpec((1, H_DIM), lambda i: (0, 0)),
          pl.BlockSpec((1, H_DIM), lambda i: (0, 0)),
          pl.BlockSpec((H_DIM, H_DIM), lambda i: (0, 0)),
      ],
      out_specs=pl.BlockSpec((blk, H_DIM), lambda i: (i, 0)),
      out_shape=jax.ShapeDtypeStruct((ea_pad, H_DIM), jnp.float32),
  )(hd, hs, w1d, w1s, w1a, b1.reshape(1, -1), w2.reshape(1, -1),
    jnp.broadcast_to(b2.reshape(1, 1), (1, H_DIM)), wmsg)


def _tc_update_body(h_ref, p0_ref, p1_ref, p2_ref, p3_ref, wh_ref, wa_ref,
                    b_ref, g_ref, bb_ref, o_ref):
  h = h_ref[...]
  agg = (p0_ref[...] + p1_ref[...]) + (p2_ref[...] + p3_ref[...])
  u = jnp.dot(h, wh_ref[...], preferred_element_type=jnp.float32)
  u += jnp.dot(agg, wa_ref[...], preferred_element_type=jnp.float32)
  u += b_ref[...] + h
  o_ref[...] = _ln_relu(u, g_ref[...], bb_ref[...])


def _tc_update(h, parts, w_upd, b_upd, g, b):
  blk = 1000
  grid = (N_NODES // blk,)
  return pl.pallas_call(
      _tc_update_body,
      grid=grid,
      in_specs=[
          pl.BlockSpec((blk, H_DIM), lambda i: (i, 0)),
          pl.BlockSpec((blk, H_DIM), lambda i: (i, 0)),
          pl.BlockSpec((blk, H_DIM), lambda i: (i, 0)),
          pl.BlockSpec((blk, H_DIM), lambda i: (i, 0)),
          pl.BlockSpec((blk, H_DIM), lambda i: (i, 0)),
          pl.BlockSpec((H_DIM, H_DIM), lambda i: (0, 0)),
          pl.BlockSpec((H_DIM, H_DIM), lambda i: (0, 0)),
          pl.BlockSpec((1, H_DIM), lambda i: (0, 0)),
          pl.BlockSpec((1, H_DIM), lambda i: (0, 0)),
          pl.BlockSpec((1, H_DIM), lambda i: (0, 0)),
      ],
      out_specs=pl.BlockSpec((blk, H_DIM), lambda i: (i, 0)),
      out_shape=jax.ShapeDtypeStruct((N_NODES, H_DIM), jnp.float32),
  )(h, parts[0], parts[1], parts[2], parts[3], w_upd[:H_DIM], w_upd[H_DIM:],
    b_upd.reshape(1, -1), g.reshape(1, -1), b.reshape(1, -1))


def _tc_head_body(h_ref, w1_ref, b1_ref, w2_ref, b2_ref, o_ref):
  u = jnp.dot(h_ref[...], w1_ref[...], preferred_element_type=jnp.float32)
  u = jax.nn.relu(u + b1_ref[...])
  o_ref[...] = jnp.dot(
      u, w2_ref[...], preferred_element_type=jnp.float32) + b2_ref[...]


def _tc_head(h, wc1, bc1, wc2, bc2):
  blk = 1000
  grid = (N_NODES // blk,)
  hh = wc1.shape[1]
  cc = wc2.shape[1]
  return pl.pallas_call(
      _tc_head_body,
      grid=grid,
      in_specs=[
          pl.BlockSpec((blk, H_DIM), lambda i: (i, 0)),
          pl.BlockSpec((H_DIM, hh), lambda i: (0, 0)),
          pl.BlockSpec((1, hh), lambda i: (0, 0)),
          pl.BlockSpec((hh, cc), lambda i: (0, 0)),
          pl.BlockSpec((1, cc), lambda i: (0, 0)),
      ],
      out_specs=pl.BlockSpec((blk, cc), lambda i: (i, 0)),
      out_shape=jax.ShapeDtypeStruct((N_NODES, cc), jnp.float32),
  )(h, wc1, bc1.reshape(1, -1), wc2, bc2.reshape(1, -1))


def kernel(x, edge_index, W_in, b_in, ln_in_g, ln_in_b, sim_W1, sim_b1,
           sim_W2, sim_b2, W_msg, W_upd, b_upd, conv_ln_g, conv_ln_b,
           Wc1, bc1, Wc2, bc2):
  n = x.shape[0]
  num_edges = edge_index.shape[1] + n
  ea_pad, chunks_per_worker = _edge_padding(num_edges)
  half = ea_pad // 2

  loops = jnp.arange(n, dtype=edge_index.dtype)
  src = jnp.concatenate([edge_index[0], loops])
  dst = jnp.concatenate([edge_index[1], loops])
  pad = ea_pad - num_edges
  src = jnp.pad(src, (0, pad)).reshape(
      2, NUM_WORKERS, chunks_per_worker, CHUNK)
  dst = jnp.pad(dst, (0, pad)).reshape(
      2, NUM_WORKERS, chunks_per_worker, CHUNK)

  gather2 = _make_sc_gather2(half, chunks_per_worker)
  scatter_add = _make_sc_scatter_add(half, chunks_per_worker)

  h = _tc_input(x, W_in, b_in, ln_in_g, ln_in_b)
  num_layers = sim_W1.shape[0]
  for l in range(num_layers):
    ha_d, ha_s = gather2(h, dst[0], src[0])
    wa = _tc_edge(ha_d, ha_s, sim_W1[l], sim_b1[l], sim_W2[l], sim_b2[l],
                  W_msg[l], num_edges, 0)
    hb_d, hb_s = gather2(h, dst[1], src[1])
    wb = _tc_edge(hb_d, hb_s, sim_W1[l], sim_b1[l], sim_W2[l], sim_b2[l],
                  W_msg[l], num_edges, half)
    pa0, pa1 = scatter_add(wa, dst[0])
    pb0, pb1 = scatter_add(wb, dst[1])
    h = _tc_update(h, (pa0[:n], pa1[:n], pb0[:n], pb1[:n]), W_upd[l],
                   b_upd[l], conv_ln_g[l], conv_ln_b[l])
  return _tc_head(h, Wc1, bc1, Wc2, bc2)


# speedup vs baseline: 1.3832x; 1.3832x over previous
import functools

import jax
import jax.numpy as jnp
from jax import lax
from jax.experimental import pallas as pl
from jax.experimental.pallas import tpu as pltpu
from jax.experimental.pallas import tpu_sc as plsc

N_NODES = 10000
H_DIM = 128

NUM_CORES = 2
NUM_SUBCORES = 16
NUM_WORKERS = NUM_CORES * NUM_SUBCORES

CHUNK = 128

N_PAD = 10368
N_CHUNKS = N_PAD // CHUNK


def _edge_padding(num_edges):
  per_round = NUM_WORKERS * CHUNK
  rounds = -(-num_edges // per_round)
  if rounds % 2 == 0:
    rounds += 1
  return rounds * per_round, rounds


def _sc_mesh():
  return plsc.VectorSubcoreMesh(
      core_axis_name="c", subcore_axis_name="s",
      num_cores=NUM_CORES, num_subcores=NUM_SUBCORES)


def _make_sc_gather2(ea_pad, chunks_per_worker):
  per_worker = chunks_per_worker * CHUNK
  assert chunks_per_worker % 2 == 1

  @functools.partial(
      pl.kernel,
      mesh=_sc_mesh(),
      out_type=(
          jax.ShapeDtypeStruct((ea_pad, H_DIM), jnp.float32),
          jax.ShapeDtypeStruct((ea_pad, H_DIM), jnp.float32),
      ),
      scratch_types=[
          pltpu.VMEM((chunks_per_worker, CHUNK), jnp.int32),
          pltpu.VMEM((chunks_per_worker, CHUNK), jnp.int32),
          pltpu.VMEM((CHUNK, H_DIM), jnp.float32),
          pltpu.VMEM((CHUNK, H_DIM), jnp.float32),
          pltpu.VMEM((CHUNK, H_DIM), jnp.float32),
          pltpu.VMEM((CHUNK, H_DIM), jnp.float32),
          pltpu.SemaphoreType.DMA,
          pltpu.SemaphoreType.DMA,
          pltpu.SemaphoreType.DMA,
          pltpu.SemaphoreType.DMA,
      ],
  )
  def gather2(h_hbm, dst_hbm, src_hbm, out_d, out_s, idx_d, idx_s,
              rows_ad, rows_as, rows_bd, rows_bs,
              sem_ad, sem_as, sem_bd, sem_bs):
    wid = lax.axis_index("s") * NUM_CORES + lax.axis_index("c")
    pltpu.sync_copy(dst_hbm.at[wid], idx_d)
    pltpu.sync_copy(src_hbm.at[wid], idx_s)

    def fire(c, rows, sem, idx):
      pltpu.async_copy(h_hbm.at[idx.at[c]], rows, sem)

    def wait(c, rows, sem, idx):
      pltpu.make_async_copy(h_hbm.at[idx.at[c]], rows, sem).wait()

    def write(c, rows, out):
      pltpu.sync_copy(
          rows, out.at[pl.ds(wid * per_worker + c * CHUNK, CHUNK)])

    fire(0, rows_ad, sem_ad, idx_d)
    fire(0, rows_as, sem_as, idx_s)

    def body(t, _):
      c0 = 2 * t
      c1 = c0 + 1
      c2 = c0 + 2
      fire(c1, rows_bd, sem_bd, idx_d)
      fire(c1, rows_bs, sem_bs, idx_s)
      wait(c0, rows_ad, sem_ad, idx_d)
      wait(c0, rows_as, sem_as, idx_s)
      write(c0, rows_ad, out_d)
      write(c0, rows_as, out_s)
      fire(c2, rows_ad, sem_ad, idx_d)
      fire(c2, rows_as, sem_as, idx_s)
      wait(c1, rows_bd, sem_bd, idx_d)
      wait(c1, rows_bs, sem_bs, idx_s)
      write(c1, rows_bd, out_d)
      write(c1, rows_bs, out_s)
      return 0

    lax.fori_loop(0, (chunks_per_worker - 1) // 2, body, 0)
    c_last = chunks_per_worker - 1
    wait(c_last, rows_ad, sem_ad, idx_d)
    wait(c_last, rows_as, sem_as, idx_s)
    write(c_last, rows_ad, out_d)
    write(c_last, rows_as, out_s)

  return gather2


def _make_sc_scatter_add(ea_pad, chunks_per_worker):
  per_worker = chunks_per_worker * CHUNK

  @functools.partial(
      pl.kernel,
      mesh=_sc_mesh(),
      out_type=(
          jax.ShapeDtypeStruct((N_PAD, H_DIM), jnp.float32),
          jax.ShapeDtypeStruct((N_PAD, H_DIM), jnp.float32),
      ),
      scratch_types=[
          pltpu.VMEM((chunks_per_worker, CHUNK), jnp.int32),
          pltpu.VMEM((CHUNK, H_DIM), jnp.float32),
          pltpu.VMEM((CHUNK, H_DIM), jnp.float32),
          pltpu.VMEM_SHARED((N_PAD, H_DIM), jnp.float32),
          pltpu.SemaphoreType.DMA,
          pltpu.SemaphoreType.DMA,
      ],
  )
  def scatter_add(w_hbm, dst_hbm, out0, out1, idx_v, rows_a, rows_b, acc_sh,
                  sem_a, sem_b):
    cid = lax.axis_index("c")
    sid = lax.axis_index("s")
    wid = sid * NUM_CORES + cid

    zvec = jnp.zeros((16,), jnp.float32)

    def zero_body(i, _):
      rows_a[i // 8, pl.ds((i % 8) * 16, 16)] = zvec
      return 0

    lax.fori_loop(0, CHUNK * 8, zero_body, 0)

    def zero_acc(t, _):
      j = sid + t * NUM_SUBCORES

      @pl.when(j < N_CHUNKS)
      def _():
        pltpu.sync_copy(rows_a, acc_sh.at[pl.ds(j * CHUNK, CHUNK)])

      return 0

    lax.fori_loop(0, -(-N_CHUNKS // NUM_SUBCORES), zero_acc, 0)
    plsc.subcore_barrier()

    pltpu.sync_copy(dst_hbm.at[wid], idx_v)

    def fire(c, rows, sem):
      base = wid * per_worker + c * CHUNK
      pltpu.async_copy(w_hbm.at[pl.ds(base, CHUNK)], rows, sem)

    def wait(c, rows, sem):
      base = wid * per_worker + c * CHUNK
      pltpu.make_async_copy(w_hbm.at[pl.ds(base, CHUNK)], rows, sem).wait()

    def add(c, rows):
      pltpu.sync_copy(rows, acc_sh.at[idx_v.at[c]], add=True)

    fire(0, rows_a, sem_a)

    def body(t, _):
      c0 = 2 * t
      c1 = c0 + 1
      c2 = c0 + 2
      fire(c1, rows_b, sem_b)
      wait(c0, rows_a, sem_a)
      add(c0, rows_a)
      fire(c2, rows_a, sem_a)
      wait(c1, rows_b, sem_b)
      add(c1, rows_b)
      return 0

    lax.fori_loop(0, (chunks_per_worker - 1) // 2, body, 0)
    c_last = chunks_per_worker - 1
    wait(c_last, rows_a, sem_a)
    add(c_last, rows_a)
    plsc.subcore_barrier()

    def write_out(t, _):
      j = sid + t * NUM_SUBCORES

      @pl.when(j < N_CHUNKS)
      def _():
        pltpu.sync_copy(acc_sh.at[pl.ds(j * CHUNK, CHUNK)], rows_a)

        @pl.when(cid == 0)
        def _():
          pltpu.sync_copy(rows_a, out0.at[pl.ds(j * CHUNK, CHUNK)])

        @pl.when(cid == 1)
        def _():
          pltpu.sync_copy(rows_a, out1.at[pl.ds(j * CHUNK, CHUNK)])

      return 0

    lax.fori_loop(0, -(-N_CHUNKS // NUM_SUBCORES), write_out, 0)

  return scatter_add


def _ln_relu(u, g, b):
  mu = jnp.mean(u, axis=-1, keepdims=True)
  var = jnp.mean((u - mu) ** 2, axis=-1, keepdims=True)
  return jax.nn.relu((u - mu) * lax.rsqrt(var + 1e-5) * g + b)


def _tc_input_body(x_ref, w_ref, b_ref, g_ref, bb_ref, o_ref):
  u = jnp.dot(x_ref[...], w_ref[...], preferred_element_type=jnp.float32)
  o_ref[...] = _ln_relu(u + b_ref[...], g_ref[...], bb_ref[...])


def _tc_input(x, w_in, b_in, g, b):
  blk = 1000
  grid = (N_NODES // blk,)
  return pl.pallas_call(
      _tc_input_body,
      grid=grid,
      in_specs=[
          pl.BlockSpec((blk, H_DIM), lambda i: (i, 0)),
          pl.BlockSpec((H_DIM, H_DIM), lambda i: (0, 0)),
          pl.BlockSpec((1, H_DIM), lambda i: (0, 0)),
          pl.BlockSpec((1, H_DIM), lambda i: (0, 0)),
          pl.BlockSpec((1, H_DIM), lambda i: (0, 0)),
      ],
      out_specs=pl.BlockSpec((blk, H_DIM), lambda i: (i, 0)),
      out_shape=jax.ShapeDtypeStruct((N_NODES, H_DIM), jnp.float32),
  )(x, w_in, b_in.reshape(1, -1), g.reshape(1, -1), b.reshape(1, -1))


def _tc_edge_body(num_edges, blk, offset, hd_ref, hs_ref, w1d_ref, w1s_ref,
                  w1a_ref, b1_ref, w2_ref, b2_ref, wmsg_ref, o_ref):
  hd = hd_ref[...]
  hs = hs_ref[...]
  d = jnp.abs(hd - hs).astype(jnp.bfloat16)
  hd16 = hd.astype(jnp.bfloat16)
  hs16 = hs.astype(jnp.bfloat16)
  hidden = jnp.dot(hd16, w1d_ref[...], preferred_element_type=jnp.float32)
  hidden += jnp.dot(hs16, w1s_ref[...], preferred_element_type=jnp.float32)
  hidden += jnp.dot(d, w1a_ref[...], preferred_element_type=jnp.float32)
  hidden = jax.nn.relu(hidden + b1_ref[...])
  s = jnp.sum(hidden * w2_ref[...], axis=-1, keepdims=True) + b2_ref[0, :1]
  score = jax.nn.sigmoid(s)
  msg = jnp.dot(hs16, wmsg_ref[...], preferred_element_type=jnp.float32)
  rows = (offset + pl.program_id(0) * blk
          + lax.broadcasted_iota(jnp.int32, (blk, 1), 0))
  o_ref[...] = jnp.where(rows < num_edges, score * msg, 0.0)


def _tc_edge(hd, hs, w1, b1, w2, b2, wmsg, num_edges, offset):
  ea_pad = hd.shape[0]
  blk = 512
  grid = (ea_pad // blk,)
  w1d = w1[:H_DIM].astype(jnp.bfloat16)
  w1s = w1[H_DIM:2 * H_DIM].astype(jnp.bfloat16)
  w1a = w1[2 * H_DIM:].astype(jnp.bfloat16)
  wmsg = wmsg.astype(jnp.bfloat16)
  return pl.pallas_call(
      functools.partial(_tc_edge_body, num_edges, blk, offset),
      grid=grid,
      in_specs=[
          pl.BlockSpec((blk, H_DIM), lambda i: (i, 0)),
          pl.BlockSpec((blk, H_DIM), lambda i: (i, 0)),
          pl.BlockSpec((H_DIM, H_DIM), lambda i: (0, 0)),
          pl.BlockSpec((H_DIM, H_DIM), lambda i: (0, 0)),
          pl.BlockSpec((H_DIM, H_DIM), lambda i: (0, 0)),
          pl.BlockSpec((1, H_DIM), lambda i: (0, 0)),
          pl.BlockSpec((1, H_DIM), lambda i: (0, 0)),
          pl.BlockSpec((1, H_DIM), lambda i: (0, 0)),
          pl.BlockSpec((H_DIM, H_DIM), lambda i: (0, 0)),
      ],
      out_specs=pl.BlockSpec((blk, H_DIM), lambda i: (i, 0)),
      out_shape=jax.ShapeDtypeStruct((ea_pad, H_DIM), jnp.float32),
  )(hd, hs, w1d, w1s, w1a, b1.reshape(1, -1), w2.reshape(1, -1),
    jnp.broadcast_to(b2.reshape(1, 1), (1, H_DIM)), wmsg)


def _tc_update_body(h_ref, p0_ref, p1_ref, wh_ref, wa_ref,
                    b_ref, g_ref, bb_ref, o_ref):
  h = h_ref[...]
  agg = p0_ref[...] + p1_ref[...]
  u = jnp.dot(h, wh_ref[...], preferred_element_type=jnp.float32)
  u += jnp.dot(agg, wa_ref[...], preferred_element_type=jnp.float32)
  u += b_ref[...] + h
  o_ref[...] = _ln_relu(u, g_ref[...], bb_ref[...])


def _tc_update(h, parts, w_upd, b_upd, g, b):
  blk = 1000
  grid = (N_NODES // blk,)
  return pl.pallas_call(
      _tc_update_body,
      grid=grid,
      in_specs=[
          pl.BlockSpec((blk, H_DIM), lambda i: (i, 0)),
          pl.BlockSpec((blk, H_DIM), lambda i: (i, 0)),
          pl.BlockSpec((blk, H_DIM), lambda i: (i, 0)),
          pl.BlockSpec((H_DIM, H_DIM), lambda i: (0, 0)),
          pl.BlockSpec((H_DIM, H_DIM), lambda i: (0, 0)),
          pl.BlockSpec((1, H_DIM), lambda i: (0, 0)),
          pl.BlockSpec((1, H_DIM), lambda i: (0, 0)),
          pl.BlockSpec((1, H_DIM), lambda i: (0, 0)),
      ],
      out_specs=pl.BlockSpec((blk, H_DIM), lambda i: (i, 0)),
      out_shape=jax.ShapeDtypeStruct((N_NODES, H_DIM), jnp.float32),
  )(h, parts[0], parts[1], w_upd[:H_DIM], w_upd[H_DIM:],
    b_upd.reshape(1, -1), g.reshape(1, -1), b.reshape(1, -1))


def _tc_head_body(h_ref, w1_ref, b1_ref, w2_ref, b2_ref, o_ref):
  u = jnp.dot(h_ref[...], w1_ref[...], preferred_element_type=jnp.float32)
  u = jax.nn.relu(u + b1_ref[...])
  o_ref[...] = jnp.dot(
      u, w2_ref[...], preferred_element_type=jnp.float32) + b2_ref[...]


def _tc_head(h, wc1, bc1, wc2, bc2):
  blk = 1000
  grid = (N_NODES // blk,)
  hh = wc1.shape[1]
  cc = wc2.shape[1]
  return pl.pallas_call(
      _tc_head_body,
      grid=grid,
      in_specs=[
          pl.BlockSpec((blk, H_DIM), lambda i: (i, 0)),
          pl.BlockSpec((H_DIM, hh), lambda i: (0, 0)),
          pl.BlockSpec((1, hh), lambda i: (0, 0)),
          pl.BlockSpec((hh, cc), lambda i: (0, 0)),
          pl.BlockSpec((1, cc), lambda i: (0, 0)),
      ],
      out_specs=pl.BlockSpec((blk, cc), lambda i: (i, 0)),
      out_shape=jax.ShapeDtypeStruct((N_NODES, cc), jnp.float32),
  )(h, wc1, bc1.reshape(1, -1), wc2, bc2.reshape(1, -1))


def kernel(x, edge_index, W_in, b_in, ln_in_g, ln_in_b, sim_W1, sim_b1,
           sim_W2, sim_b2, W_msg, W_upd, b_upd, conv_ln_g, conv_ln_b,
           Wc1, bc1, Wc2, bc2):
  n = x.shape[0]
  num_edges = edge_index.shape[1] + n
  ea_pad, chunks_per_worker = _edge_padding(num_edges)

  loops = jnp.arange(n, dtype=edge_index.dtype)
  src = jnp.concatenate([edge_index[0], loops])
  dst = jnp.concatenate([edge_index[1], loops])
  pad = ea_pad - num_edges
  src = jnp.pad(src, (0, pad)).reshape(NUM_WORKERS, chunks_per_worker, CHUNK)
  dst = jnp.pad(dst, (0, pad)).reshape(NUM_WORKERS, chunks_per_worker, CHUNK)

  gather2 = _make_sc_gather2(ea_pad, chunks_per_worker)
  scatter_add = _make_sc_scatter_add(ea_pad, chunks_per_worker)

  h = _tc_input(x, W_in, b_in, ln_in_g, ln_in_b)
  num_layers = sim_W1.shape[0]
  for l in range(num_layers):
    hd, hs = gather2(h, dst, src)
    weighted = _tc_edge(hd, hs, sim_W1[l], sim_b1[l], sim_W2[l], sim_b2[l],
                        W_msg[l], num_edges, 0)
    p0, p1 = scatter_add(weighted, dst)
    h = _tc_update(h, (p0[:n], p1[:n]), W_upd[l], b_upd[l], conv_ln_g[l],
                   conv_ln_b[l])
  return _tc_head(h, Wc1, bc1, Wc2, bc2)


# speedup vs baseline: 1.5440x; 1.1163x over previous
import functools

import jax
import jax.numpy as jnp
from jax import lax
from jax.experimental import pallas as pl
from jax.experimental.pallas import tpu as pltpu
from jax.experimental.pallas import tpu_sc as plsc

N_NODES = 10000
H_DIM = 128

NUM_CORES = 2
NUM_SUBCORES = 16
NUM_WORKERS = NUM_CORES * NUM_SUBCORES

CHUNK = 128

N_PAD = 10368
N_CHUNKS = N_PAD // CHUNK


def _edge_padding(num_edges):
  per_round = NUM_SUBCORES * CHUNK
  rounds = -(-num_edges // per_round)
  if rounds % 2 == 1:
    rounds += 1
  return rounds * per_round, rounds


def _sc_mesh():
  return plsc.VectorSubcoreMesh(
      core_axis_name="c", subcore_axis_name="s",
      num_cores=NUM_CORES, num_subcores=NUM_SUBCORES)


def _make_sc_gather2(ea_pad, ch_tot):
  chunks = ch_tot // NUM_CORES
  per_worker = chunks * CHUNK
  assert chunks % 2 == 1

  @functools.partial(
      pl.kernel,
      mesh=_sc_mesh(),
      out_type=(
          jax.ShapeDtypeStruct((ea_pad, H_DIM), jnp.float32),
          jax.ShapeDtypeStruct((ea_pad, H_DIM), jnp.float32),
      ),
      scratch_types=[
          pltpu.VMEM((chunks, CHUNK), jnp.int32),
          pltpu.VMEM((chunks, CHUNK), jnp.int32),
          pltpu.VMEM((CHUNK, H_DIM), jnp.float32),
          pltpu.VMEM((CHUNK, H_DIM), jnp.float32),
          pltpu.VMEM((CHUNK, H_DIM), jnp.float32),
          pltpu.VMEM((CHUNK, H_DIM), jnp.float32),
          pltpu.SemaphoreType.DMA,
          pltpu.SemaphoreType.DMA,
          pltpu.SemaphoreType.DMA,
          pltpu.SemaphoreType.DMA,
      ],
  )
  def gather2(h_hbm, dst_hbm, src_hbm, out_d, out_s, idx_d, idx_s,
              rows_ad, rows_as, rows_bd, rows_bs,
              sem_ad, sem_as, sem_bd, sem_bs):
    wid = lax.axis_index("s") * NUM_CORES + lax.axis_index("c")
    pltpu.sync_copy(dst_hbm.at[wid], idx_d)
    pltpu.sync_copy(src_hbm.at[wid], idx_s)

    def fire(c, rows, sem, idx):
      pltpu.async_copy(h_hbm.at[idx.at[c]], rows, sem)

    def wait(c, rows, sem, idx):
      pltpu.make_async_copy(h_hbm.at[idx.at[c]], rows, sem).wait()

    def write(c, rows, out):
      pltpu.sync_copy(
          rows, out.at[pl.ds(wid * per_worker + c * CHUNK, CHUNK)])

    fire(0, rows_ad, sem_ad, idx_d)
    fire(0, rows_as, sem_as, idx_s)

    def body(t, _):
      c0 = 2 * t
      c1 = c0 + 1
      c2 = c0 + 2
      fire(c1, rows_bd, sem_bd, idx_d)
      fire(c1, rows_bs, sem_bs, idx_s)
      wait(c0, rows_ad, sem_ad, idx_d)
      wait(c0, rows_as, sem_as, idx_s)
      write(c0, rows_ad, out_d)
      write(c0, rows_as, out_s)
      fire(c2, rows_ad, sem_ad, idx_d)
      fire(c2, rows_as, sem_as, idx_s)
      wait(c1, rows_bd, sem_bd, idx_d)
      wait(c1, rows_bs, sem_bs, idx_s)
      write(c1, rows_bd, out_d)
      write(c1, rows_bs, out_s)
      return 0

    lax.fori_loop(0, (chunks - 1) // 2, body, 0)
    c_last = chunks - 1
    wait(c_last, rows_ad, sem_ad, idx_d)
    wait(c_last, rows_as, sem_as, idx_s)
    write(c_last, rows_ad, out_d)
    write(c_last, rows_as, out_s)

  return gather2


def _make_sc_scatter_add(ea_pad, chunks_per_worker):
  per_worker = chunks_per_worker * CHUNK

  @functools.partial(
      pl.kernel,
      mesh=_sc_mesh(),
      out_type=(
          jax.ShapeDtypeStruct((N_PAD, H_DIM), jnp.float32),
          jax.ShapeDtypeStruct((N_PAD, H_DIM), jnp.float32),
      ),
      scratch_types=[
          pltpu.VMEM((chunks_per_worker, CHUNK), jnp.int32),
          pltpu.VMEM((CHUNK, H_DIM), jnp.float32),
          pltpu.VMEM((CHUNK, H_DIM), jnp.float32),
          pltpu.VMEM_SHARED((N_PAD, H_DIM), jnp.float32),
          pltpu.SemaphoreType.DMA,
          pltpu.SemaphoreType.DMA,
      ],
  )
  def scatter_add(w_hbm, dst_hbm, out0, out1, idx_v, rows_a, rows_b, acc_sh,
                  sem_a, sem_b):
    cid = lax.axis_index("c")
    sid = lax.axis_index("s")
    wid = sid * NUM_CORES + cid

    zvec = jnp.zeros((16,), jnp.float32)

    def zero_body(i, _):
      rows_a[i // 8, pl.ds((i % 8) * 16, 16)] = zvec
      return 0

    lax.fori_loop(0, CHUNK * 8, zero_body, 0)

    def zero_acc(t, _):
      j = sid + t * NUM_SUBCORES

      @pl.when(j < N_CHUNKS)
      def _():
        pltpu.sync_copy(rows_a, acc_sh.at[pl.ds(j * CHUNK, CHUNK)])

      return 0

    lax.fori_loop(0, -(-N_CHUNKS // NUM_SUBCORES), zero_acc, 0)
    plsc.subcore_barrier()

    pltpu.sync_copy(dst_hbm.at[wid], idx_v)

    def fire(c, rows, sem):
      base = wid * per_worker + c * CHUNK
      pltpu.async_copy(w_hbm.at[pl.ds(base, CHUNK)], rows, sem)

    def wait(c, rows, sem):
      base = wid * per_worker + c * CHUNK
      pltpu.make_async_copy(w_hbm.at[pl.ds(base, CHUNK)], rows, sem).wait()

    def add(c, rows):
      pltpu.sync_copy(rows, acc_sh.at[idx_v.at[c]], add=True)

    fire(0, rows_a, sem_a)

    def body(t, _):
      c0 = 2 * t
      c1 = c0 + 1
      c2 = c0 + 2
      fire(c1, rows_b, sem_b)
      wait(c0, rows_a, sem_a)
      add(c0, rows_a)
      fire(c2, rows_a, sem_a)
      wait(c1, rows_b, sem_b)
      add(c1, rows_b)
      return 0

    lax.fori_loop(0, (chunks_per_worker - 1) // 2, body, 0)
    c_last = chunks_per_worker - 1
    wait(c_last, rows_a, sem_a)
    add(c_last, rows_a)
    plsc.subcore_barrier()

    def write_out(t, _):
      j = sid + t * NUM_SUBCORES

      @pl.when(j < N_CHUNKS)
      def _():
        pltpu.sync_copy(acc_sh.at[pl.ds(j * CHUNK, CHUNK)], rows_a)

        @pl.when(cid == 0)
        def _():
          pltpu.sync_copy(rows_a, out0.at[pl.ds(j * CHUNK, CHUNK)])

        @pl.when(cid == 1)
        def _():
          pltpu.sync_copy(rows_a, out1.at[pl.ds(j * CHUNK, CHUNK)])

      return 0

    lax.fori_loop(0, -(-N_CHUNKS // NUM_SUBCORES), write_out, 0)

  return scatter_add


def _ln_relu(u, g, b):
  mu = jnp.mean(u, axis=-1, keepdims=True)
  var = jnp.mean((u - mu) ** 2, axis=-1, keepdims=True)
  return jax.nn.relu((u - mu) * lax.rsqrt(var + 1e-5) * g + b)


def _tc_input_body(x_ref, w_ref, b_ref, g_ref, bb_ref, o_ref):
  u = jnp.dot(x_ref[...], w_ref[...], preferred_element_type=jnp.float32)
  o_ref[...] = _ln_relu(u + b_ref[...], g_ref[...], bb_ref[...])


def _tc_input(x, w_in, b_in, g, b):
  blk = 1000
  grid = (N_NODES // blk,)
  return pl.pallas_call(
      _tc_input_body,
      grid=grid,
      in_specs=[
          pl.BlockSpec((blk, H_DIM), lambda i: (i, 0)),
          pl.BlockSpec((H_DIM, H_DIM), lambda i: (0, 0)),
          pl.BlockSpec((1, H_DIM), lambda i: (0, 0)),
          pl.BlockSpec((1, H_DIM), lambda i: (0, 0)),
          pl.BlockSpec((1, H_DIM), lambda i: (0, 0)),
      ],
      out_specs=pl.BlockSpec((blk, H_DIM), lambda i: (i, 0)),
      out_shape=jax.ShapeDtypeStruct((N_NODES, H_DIM), jnp.float32),
  )(x, w_in, b_in.reshape(1, -1), g.reshape(1, -1), b.reshape(1, -1))


def _tc_edge_body(num_edges, blk, offset, hd_ref, hs_ref, w1d_ref, w1s_ref,
                  w1a_ref, b1_ref, w2_ref, b2_ref, wmsg_ref, o_ref):
  hd = hd_ref[...]
  hs = hs_ref[...]
  d = jnp.abs(hd - hs).astype(jnp.bfloat16)
  hd16 = hd.astype(jnp.bfloat16)
  hs16 = hs.astype(jnp.bfloat16)
  hidden = jnp.dot(hd16, w1d_ref[...], preferred_element_type=jnp.float32)
  hidden += jnp.dot(hs16, w1s_ref[...], preferred_element_type=jnp.float32)
  hidden += jnp.dot(d, w1a_ref[...], preferred_element_type=jnp.float32)
  hidden = jax.nn.relu(hidden + b1_ref[...])
  s = jnp.sum(hidden * w2_ref[...], axis=-1, keepdims=True) + b2_ref[0, :1]
  score = jax.nn.sigmoid(s)
  msg = jnp.dot(hs16, wmsg_ref[...], preferred_element_type=jnp.float32)
  rows = (offset + pl.program_id(0) * blk
          + lax.broadcasted_iota(jnp.int32, (blk, 1), 0))
  o_ref[...] = jnp.where(rows < num_edges, score * msg, 0.0)


def _tc_edge(hd, hs, w1, b1, w2, b2, wmsg, num_edges, offset):
  ea_pad = hd.shape[0]
  blk = 1024
  grid = (ea_pad // blk,)
  w1d = w1[:H_DIM].astype(jnp.bfloat16)
  w1s = w1[H_DIM:2 * H_DIM].astype(jnp.bfloat16)
  w1a = w1[2 * H_DIM:].astype(jnp.bfloat16)
  wmsg = wmsg.astype(jnp.bfloat16)
  return pl.pallas_call(
      functools.partial(_tc_edge_body, num_edges, blk, offset),
      grid=grid,
      in_specs=[
          pl.BlockSpec((blk, H_DIM), lambda i: (i, 0)),
          pl.BlockSpec((blk, H_DIM), lambda i: (i, 0)),
          pl.BlockSpec((H_DIM, H_DIM), lambda i: (0, 0)),
          pl.BlockSpec((H_DIM, H_DIM), lambda i: (0, 0)),
          pl.BlockSpec((H_DIM, H_DIM), lambda i: (0, 0)),
          pl.BlockSpec((1, H_DIM), lambda i: (0, 0)),
          pl.BlockSpec((1, H_DIM), lambda i: (0, 0)),
          pl.BlockSpec((1, H_DIM), lambda i: (0, 0)),
          pl.BlockSpec((H_DIM, H_DIM), lambda i: (0, 0)),
      ],
      out_specs=pl.BlockSpec((blk, H_DIM), lambda i: (i, 0)),
      out_shape=jax.ShapeDtypeStruct((ea_pad, H_DIM), jnp.float32),
  )(hd, hs, w1d, w1s, w1a, b1.reshape(1, -1), w2.reshape(1, -1),
    jnp.broadcast_to(b2.reshape(1, 1), (1, H_DIM)), wmsg)


def _tc_update_body(h_ref, p0_ref, p1_ref, wh_ref, wa_ref,
                    b_ref, g_ref, bb_ref, o_ref):
  h = h_ref[...]
  agg = p0_ref[...] + p1_ref[...]
  u = jnp.dot(h, wh_ref[...], preferred_element_type=jnp.float32)
  u += jnp.dot(agg, wa_ref[...], preferred_element_type=jnp.float32)
  u += b_ref[...] + h
  o_ref[...] = _ln_relu(u, g_ref[...], bb_ref[...])


def _tc_update(h, parts, w_upd, b_upd, g, b):
  blk = 1000
  grid = (N_NODES // blk,)
  return pl.pallas_call(
      _tc_update_body,
      grid=grid,
      in_specs=[
          pl.BlockSpec((blk, H_DIM), lambda i: (i, 0)),
          pl.BlockSpec((blk, H_DIM), lambda i: (i, 0)),
          pl.BlockSpec((blk, H_DIM), lambda i: (i, 0)),
          pl.BlockSpec((H_DIM, H_DIM), lambda i: (0, 0)),
          pl.BlockSpec((H_DIM, H_DIM), lambda i: (0, 0)),
          pl.BlockSpec((1, H_DIM), lambda i: (0, 0)),
          pl.BlockSpec((1, H_DIM), lambda i: (0, 0)),
          pl.BlockSpec((1, H_DIM), lambda i: (0, 0)),
      ],
      out_specs=pl.BlockSpec((blk, H_DIM), lambda i: (i, 0)),
      out_shape=jax.ShapeDtypeStruct((N_NODES, H_DIM), jnp.float32),
  )(h, parts[0], parts[1], w_upd[:H_DIM], w_upd[H_DIM:],
    b_upd.reshape(1, -1), g.reshape(1, -1), b.reshape(1, -1))


def _tc_head_body(h_ref, w1_ref, b1_ref, w2_ref, b2_ref, o_ref):
  u = jnp.dot(h_ref[...], w1_ref[...], preferred_element_type=jnp.float32)
  u = jax.nn.relu(u + b1_ref[...])
  o_ref[...] = jnp.dot(
      u, w2_ref[...], preferred_element_type=jnp.float32) + b2_ref[...]


def _tc_head(h, wc1, bc1, wc2, bc2):
  blk = 1000
  grid = (N_NODES // blk,)
  hh = wc1.shape[1]
  cc = wc2.shape[1]
  return pl.pallas_call(
      _tc_head_body,
      grid=grid,
      in_specs=[
          pl.BlockSpec((blk, H_DIM), lambda i: (i, 0)),
          pl.BlockSpec((H_DIM, hh), lambda i: (0, 0)),
          pl.BlockSpec((1, hh), lambda i: (0, 0)),
          pl.BlockSpec((hh, cc), lambda i: (0, 0)),
          pl.BlockSpec((1, cc), lambda i: (0, 0)),
      ],
      out_specs=pl.BlockSpec((blk, cc), lambda i: (i, 0)),
      out_shape=jax.ShapeDtypeStruct((N_NODES, cc), jnp.float32),
  )(h, wc1, bc1.reshape(1, -1), wc2, bc2.reshape(1, -1))


def kernel(x, edge_index, W_in, b_in, ln_in_g, ln_in_b, sim_W1, sim_b1,
           sim_W2, sim_b2, W_msg, W_upd, b_upd, conv_ln_g, conv_ln_b,
           Wc1, bc1, Wc2, bc2):
  n = x.shape[0]
  num_edges = edge_index.shape[1] + n
  ea_pad, ch_tot = _edge_padding(num_edges)
  total_chunks = ea_pad // CHUNK

  loops = jnp.arange(n, dtype=edge_index.dtype)
  src = jnp.concatenate([edge_index[0], loops])
  dst = jnp.concatenate([edge_index[1], loops])
  pad = ea_pad - num_edges
  src_flat = jnp.pad(src, (0, pad))
  dst_flat = jnp.pad(dst, (0, pad))
  chw = total_chunks // NUM_WORKERS
  src_g = src_flat.reshape(NUM_WORKERS, chw, CHUNK)
  dst_g = dst_flat.reshape(NUM_WORKERS, chw, CHUNK)
  dst_s = dst_g

  gather2 = _make_sc_gather2(ea_pad, ch_tot)
  scatter_add = _make_sc_scatter_add(ea_pad, chw)

  h = _tc_input(x, W_in, b_in, ln_in_g, ln_in_b)
  num_layers = sim_W1.shape[0]
  for l in range(num_layers):
    hd, hs = gather2(h, dst_g, src_g)
    weighted = _tc_edge(hd, hs, sim_W1[l], sim_b1[l], sim_W2[l], sim_b2[l],
                        W_msg[l], num_edges, 0)
    p0, p1 = scatter_add(weighted, dst_s)
    h = _tc_update(h, (p0[:n], p1[:n]), W_upd[l], b_upd[l], conv_ln_g[l],
                   conv_ln_b[l])
  return _tc_head(h, Wc1, bc1, Wc2, bc2)
